# baseline (speedup 1.0000x reference)
import functools

import jax
import jax.numpy as jnp
from jax import lax
from jax.experimental import pallas as pl
from jax.experimental.pallas import tpu as pltpu
from jax.experimental.pallas import tpu_sc as plsc


def kernel(x0, x1, indices):
    E, D = x1.shape
    N = x0.shape[0]
    info = plsc.get_sparse_core_info()
    NC, NS = info.num_cores, info.num_subcores
    NW = NC * NS
    assert E % NW == 0
    e_per_w = E // NW
    B = 40
    assert e_per_w % B == 0
    niter = e_per_w // B
    n_per_s = (N // NS) // 8 * 8
    n_tail = N - NS * n_per_s

    mesh = plsc.VectorSubcoreMesh(core_axis_name="c", subcore_axis_name="s")

    @functools.partial(
        pl.kernel,
        mesh=mesh,
        out_type=jax.ShapeDtypeStruct((E, D), jnp.float32),
        scratch_types=[
            pltpu.VMEM_SHARED((N, D), jnp.float32),
            pltpu.VMEM((e_per_w,), jnp.int32),
            pltpu.VMEM((B, D), jnp.float32),
            pltpu.VMEM((B, D), jnp.float32),
            pltpu.VMEM((B, D), jnp.float32),
            pltpu.VMEM((B, D), jnp.float32),
            pltpu.VMEM((B, D), jnp.float32),
            pltpu.VMEM((B, D), jnp.float32),
            pltpu.SemaphoreType.DMA,
            pltpu.SemaphoreType.DMA,
            pltpu.SemaphoreType.DMA,
            pltpu.SemaphoreType.DMA,
            pltpu.SemaphoreType.DMA,
            pltpu.SemaphoreType.DMA,
        ],
    )
    def run(x0_hbm, x1_hbm, idx_hbm, out_hbm,
            x0_sh, idx_v, w0, w1, y0, y1, o0, o1,
            g0, g1, p0, p1, q0, q1):
        sid = lax.axis_index("s")
        wid = sid * NC + lax.axis_index("c")
        base = wid * e_per_w
        pltpu.sync_copy(x0_hbm.at[pl.ds(sid * n_per_s, n_per_s)],
                        x0_sh.at[pl.ds(sid * n_per_s, n_per_s)])
        if n_tail:
            @pl.when(sid == NS - 1)
            def _():
                pltpu.sync_copy(x0_hbm.at[pl.ds(NS * n_per_s, n_tail)],
                                x0_sh.at[pl.ds(NS * n_per_s, n_tail)])
        pltpu.sync_copy(idx_hbm.at[pl.ds(base, e_per_w)], idx_v)
        plsc.subcore_barrier()

        wbufs = (w0, w1)
        ybufs = (y0, y1)
        obufs = (o0, o1)
        gsems = (g0, g1)
        xsems = (p0, p1)
        osems = (q0, q1)

        def issue_inputs(i, s):
            pltpu.async_copy(
                x0_sh.at[idx_v.at[pl.ds(i * B, B)]], wbufs[s], gsems[s])
            pltpu.async_copy(
                x1_hbm.at[pl.ds(base + i * B, B)], ybufs[s], xsems[s])

        issue_inputs(0, 0)
        issue_inputs(1, 1)

        def step(i, s):
            w, y, o = wbufs[s], ybufs[s], obufs[s]
            pltpu.make_async_copy(x1_hbm.at[pl.ds(0, B)], w, gsems[s]).wait()
            pltpu.make_async_copy(x1_hbm.at[pl.ds(0, B)], y, xsems[s]).wait()

            @pl.when(i >= 2)
            def _():
                pltpu.make_async_copy(o, out_hbm.at[pl.ds(0, B)], osems[s]).wait()

            @plsc.parallel_loop(0, B, unroll=4)
            def row(r):
                for c in range(D // 16):
                    sl = pl.ds(c * 16, 16)
                    o[r, sl] = w[r, sl] * y[r, sl]

            pltpu.async_copy(o, out_hbm.at[pl.ds(base + i * B, B)], osems[s])

            @pl.when(i + 2 < niter)
            def _():
                issue_inputs(i + 2, s)

        def outer(g, carry):
            step(2 * g, 0)
            step(2 * g + 1, 1)
            return carry

        lax.fori_loop(0, niter // 2, outer, 0)
        if niter % 2:
            step(niter - 1, 0)

        pltpu.make_async_copy(o0, out_hbm.at[pl.ds(0, B)], osems[0]).wait()
        pltpu.make_async_copy(o1, out_hbm.at[pl.ds(0, B)], osems[1]).wait()

    return run(x0, x1, indices)


# speedup vs baseline: 1.0690x; 1.0690x over previous
import functools

import jax
import jax.numpy as jnp
from jax import lax
from jax.experimental import pallas as pl
from jax.experimental.pallas import tpu as pltpu
from jax.experimental.pallas import tpu_sc as plsc


def kernel(x0, x1, indices):
    E, D = x1.shape
    info = plsc.get_sparse_core_info()
    NC, NS = info.num_cores, info.num_subcores
    NW = NC * NS
    assert E % NW == 0
    e_per_w = E // NW
    B = 200
    GA = 104
    GB = B - GA
    assert e_per_w % B == 0
    niter = e_per_w // B
    assert niter % 6 == 2

    mesh = plsc.VectorSubcoreMesh(core_axis_name="c", subcore_axis_name="s")

    @functools.partial(
        pl.kernel,
        mesh=mesh,
        out_type=jax.ShapeDtypeStruct((E, D), jnp.float32),
        scratch_types=(
            [pltpu.VMEM((B,), jnp.int32) for _ in range(6)]
            + [pltpu.VMEM((B, D), jnp.float32) for _ in range(2)]
            + [pltpu.VMEM((B, D), jnp.float32) for _ in range(3)]
            + [pltpu.SemaphoreType.DMA for _ in range(6)]
            + [pltpu.SemaphoreType.DMA for _ in range(2)]
            + [pltpu.SemaphoreType.DMA for _ in range(3)]
            + [pltpu.SemaphoreType.DMA for _ in range(3)]
        ),
    )
    def run(x0_hbm, x1_hbm, idx_hbm, out_hbm,
            i0, i1, i2, i3, i4, i5, w0, w1, y0, y1, y2,
            si0, si1, si2, si3, si4, si5, g0, g1, p0, p1, p2, q0, q1, q2):
        wid = lax.axis_index("s") * NC + lax.axis_index("c")
        base = wid * e_per_w

        ibufs = (i0, i1, i2, i3, i4, i5)
        wbufs = (w0, w1)
        ybufs = (y0, y1, y2)
        isems = (si0, si1, si2, si3, si4, si5)
        gsems = (g0, g1)
        xsems = (p0, p1, p2)
        osems = (q0, q1, q2)

        def issue_idx(j, s6):
            pltpu.async_copy(
                idx_hbm.at[pl.ds(base + j * B, B)], ibufs[s6], isems[s6])

        def issue_inputs(j, s2, s3, s6):
            pltpu.make_async_copy(
                idx_hbm.at[pl.ds(0, B)], ibufs[s6], isems[s6]).wait()
            pltpu.async_copy(
                x0_hbm.at[ibufs[s6].at[pl.ds(0, GA)]],
                wbufs[s2].at[pl.ds(0, GA)], gsems[s2])
            pltpu.async_copy(
                x0_hbm.at[ibufs[s6].at[pl.ds(GA, GB)]],
                wbufs[s2].at[pl.ds(GA, GB)], gsems[s2])
            pltpu.async_copy(
                x1_hbm.at[pl.ds(base + j * B, B)], ybufs[s3], xsems[s3])

        for jj in range(4):
            issue_idx(jj, jj)
        for jj in range(2):
            issue_inputs(jj, jj, jj, jj)

        def step(j, k):
            s2, s3, s6 = k % 2, k % 3, k
            w, y = wbufs[s2], ybufs[s3]
            pltpu.make_async_copy(
                x1_hbm.at[pl.ds(0, GA)], w.at[pl.ds(0, GA)], gsems[s2]).wait()
            pltpu.make_async_copy(
                x1_hbm.at[pl.ds(0, GB)], w.at[pl.ds(GA, GB)], gsems[s2]).wait()
            pltpu.make_async_copy(
                x1_hbm.at[pl.ds(0, B)], y, xsems[s3]).wait()

            @plsc.parallel_loop(0, B, unroll=8)
            def row(r):
                for c in range(D // 16):
                    sl = pl.ds(c * 16, 16)
                    y[r, sl] = w[r, sl] * y[r, sl]

            pltpu.async_copy(y, out_hbm.at[pl.ds(base + j * B, B)], osems[s3])

            @pl.when(j + 4 < niter)
            def _():
                issue_idx(j + 4, (k + 4) % 6)

            @pl.when(jnp.logical_and(j >= 1, j + 2 < niter))
            def _():
                pltpu.make_async_copy(
                    ybufs[(k + 2) % 3], out_hbm.at[pl.ds(0, B)],
                    osems[(k + 2) % 3]).wait()

            @pl.when(j + 2 < niter)
            def _():
                issue_inputs(j + 2, (k + 2) % 2, (k + 2) % 3, (k + 2) % 6)

        def outer(g, carry):
            for k in range(6):
                step(6 * g + k, k)
            return carry

        lax.fori_loop(0, (niter - 2) // 6, outer, 0)
        step(niter - 2, 0)
        step(niter - 1, 1)

        for s3 in ((niter - 3) % 3, (niter - 2) % 3, (niter - 1) % 3):
            pltpu.make_async_copy(
                ybufs[s3], out_hbm.at[pl.ds(0, B)], osems[s3]).wait()

    return run(x0, x1, indices)


# speedup vs baseline: 1.1092x; 1.0376x over previous
import functools

import jax
import jax.numpy as jnp
from jax import lax
from jax.experimental import pallas as pl
from jax.experimental.pallas import tpu as pltpu
from jax.experimental.pallas import tpu_sc as plsc


def kernel(x0, x1, indices):
    E, D = x1.shape
    info = plsc.get_sparse_core_info()
    NC, NS = info.num_cores, info.num_subcores
    NW = NC * NS
    assert E % NW == 0
    e_per_w = E // NW
    B = 80
    assert e_per_w % B == 0
    niter = e_per_w // B
    assert niter % 4 == 1

    mesh = plsc.VectorSubcoreMesh(core_axis_name="c", subcore_axis_name="s")

    @functools.partial(
        pl.kernel,
        mesh=mesh,
        out_type=jax.ShapeDtypeStruct((E, D), jnp.float32),
        scratch_types=(
            [pltpu.VMEM((e_per_w,), jnp.int32)]
            + [pltpu.VMEM((B, D), jnp.float32) for _ in range(4)]
            + [pltpu.VMEM((B, D), jnp.float32) for _ in range(4)]
            + [pltpu.VMEM((B, D), jnp.float32) for _ in range(2)]
            + [pltpu.SemaphoreType.DMA for _ in range(4)]
            + [pltpu.SemaphoreType.DMA for _ in range(4)]
            + [pltpu.SemaphoreType.DMA for _ in range(2)]
        ),
    )
    def run(x0_hbm, x1_hbm, idx_hbm, out_hbm,
            idx_v, w0, w1, w2, w3, y0, y1, y2, y3, o0, o1,
            g0, g1, g2, g3, p0, p1, p2, p3, q0, q1):
        wid = lax.axis_index("s") * NC + lax.axis_index("c")
        base = wid * e_per_w
        pltpu.sync_copy(idx_hbm.at[pl.ds(base, e_per_w)], idx_v)

        wbufs = (w0, w1, w2, w3)
        ybufs = (y0, y1, y2, y3)
        obufs = (o0, o1)
        gsems = (g0, g1, g2, g3)
        xsems = (p0, p1, p2, p3)
        osems = (q0, q1)

        def issue_inputs(j, s4):
            pltpu.async_copy(
                x0_hbm.at[idx_v.at[pl.ds(j * B, B)]], wbufs[s4], gsems[s4])
            pltpu.async_copy(
                x1_hbm.at[pl.ds(base + j * B, B)], ybufs[s4], xsems[s4])

        for jj in range(4):
            issue_inputs(jj, jj)

        def step(j, k):
            s4, s2 = k, k % 2
            w, y, o = wbufs[s4], ybufs[s4], obufs[s2]
            pltpu.make_async_copy(x1_hbm.at[pl.ds(0, B)], w, gsems[s4]).wait()
            pltpu.make_async_copy(x1_hbm.at[pl.ds(0, B)], y, xsems[s4]).wait()

            @pl.when(j >= 2)
            def _():
                pltpu.make_async_copy(o, out_hbm.at[pl.ds(0, B)], osems[s2]).wait()

            @plsc.parallel_loop(0, B, unroll=8)
            def row(r):
                for c in range(D // 16):
                    sl = pl.ds(c * 16, 16)
                    o[r, sl] = w[r, sl] * y[r, sl]

            pltpu.async_copy(o, out_hbm.at[pl.ds(base + j * B, B)], osems[s2])

            @pl.when(j + 4 < niter)
            def _():
                issue_inputs(j + 4, s4)

        def outer(g, carry):
            for k in range(4):
                step(4 * g + k, k)
            return carry

        lax.fori_loop(0, (niter - 1) // 4, outer, 0)
        step(niter - 1, 0)

        pltpu.make_async_copy(o1, out_hbm.at[pl.ds(0, B)], osems[1]).wait()
        pltpu.make_async_copy(o0, out_hbm.at[pl.ds(0, B)], osems[0]).wait()

    return run(x0, x1, indices)
